# Initial kernel scaffold; baseline (speedup 1.0000x reference)
import jax
import jax.numpy as jnp
from jax.experimental import pallas as pl


def kernel(node_logits, comb_logits, edge_index):
    raise NotImplementedError("write your pallas kernel here")


# speedup vs baseline: 2753.4138x; 2753.4138x over previous
import functools

import jax
import jax.numpy as jnp
from jax import lax
from jax.experimental import pallas as pl
from jax.experimental.pallas import tpu as pltpu
from jax.experimental.pallas import tpu_sc as plsc

_L = 16


def _entropy_tc_body(l0_ref, l1_ref, h_ref):
    l0 = l0_ref[...]
    l1 = l1_ref[...]
    m = jnp.maximum(l0, l1)
    e0 = jnp.exp(l0 - m)
    e1 = jnp.exp(l1 - m)
    tot = e0 + e1
    p0 = e0 / tot
    p1 = e1 / tot
    eps = 1e-10
    factor = 1.0 + 0.01 / (1.0 + 1 * 0)
    h = ((p0 + eps) * jnp.log(1.0 / (p0 + eps) + eps)
         + (p1 + eps) * jnp.log(1.0 / (p1 + eps) + eps))
    h_ref[...] = h * factor


def _floor16(x):
    t = x.astype(jnp.int32).astype(jnp.float32)
    return t - jnp.where(x < t, 1.0, 0.0)


def _rmw_max(ref, idx, s):
    cur = plsc.load_gather(ref, [idx])

    def cond(p):
        return jnp.any(p)

    def body(p):
        plsc.store_scatter(ref, [idx], s, mask=p)
        return s > plsc.load_gather(ref, [idx])

    lax.while_loop(cond, body, s > cur)


def _make_sc_kernels(n_nodes, n_edges):
    try:
        info = plsc.get_sparse_core_info()
        nc, ns = info.num_cores, info.num_subcores
    except ValueError:
        nc, ns = 2, 16
    nw = nc * ns
    slc = ((n_nodes + ns * _L - 1) // (ns * _L)) * _L
    n_pad = ns * slc
    chunk = ((n_edges + nw * _L - 1) // (nw * _L)) * _L
    e_pad = nw * chunk
    mesh = plsc.VectorSubcoreMesh(core_axis_name="c", subcore_axis_name="s")

    @functools.partial(
        pl.kernel,
        out_type=(
            jax.ShapeDtypeStruct((e_pad,), jnp.float32),
            jax.ShapeDtypeStruct((nc * n_pad,), jnp.float32),
        ),
        mesh=mesh,
        compiler_params=pltpu.CompilerParams(needs_layout_passes=False),
        scratch_types=[
            pltpu.VMEM((n_pad,), jnp.float32),
            pltpu.VMEM((n_pad,), jnp.float32),
            pltpu.VMEM((chunk,), jnp.int32),
            pltpu.VMEM((chunk,), jnp.int32),
            pltpu.VMEM((chunk,), jnp.float32),
            pltpu.VMEM((chunk,), jnp.float32),
            pltpu.VMEM_SHARED((ns * n_pad,), jnp.float32),
        ],
    )
    def sc1(h_all, src, dst, scores_out, nm_out, h_v, nm_v, src_v, dst_v,
            hc_v, sc_v, partials):
        cid = lax.axis_index("c")
        sid = lax.axis_index("s")
        wid = sid * nc + cid
        base = wid * chunk

        pltpu.sync_copy(h_all.at[pl.ds(0, n_pad)], h_v)
        pltpu.sync_copy(h_all.at[pl.ds(n_pad + base, chunk)], hc_v)
        pltpu.sync_copy(src.at[pl.ds(base, chunk)], src_v)
        pltpu.sync_copy(dst.at[pl.ds(base, chunk)], dst_v)

        zeros = jnp.zeros((_L,), jnp.float32)

        def zero_body(j, _):
            nm_v[pl.ds(j * _L, _L)] = zeros
            return 0

        lax.fori_loop(0, n_pad // _L, zero_body, 0)

        iota = lax.iota(jnp.int32, _L)

        def edge_body(j, _):
            sl = pl.ds(j * _L, _L)
            si = src_v[sl]
            di = dst_v[sl]
            hc = hc_v[sl]
            hs = plsc.load_gather(h_v, [si])
            hd = plsc.load_gather(h_v, [di])
            a = hs - hc
            b = hd - hc
            fa = _floor16(a)
            fb = _floor16(b)
            s = (2.0 + a) * (2.0 + b) * ((1.0 + fa) * (1.0 + fb))
            lane = base + j * _L + iota
            s = jnp.where(lane < n_edges, s, 0.0)
            sc_v[sl] = s
            _rmw_max(nm_v, si, s)
            _rmw_max(nm_v, di, s)
            return 0

        lax.fori_loop(0, chunk // _L, edge_body, 0)

        pltpu.sync_copy(sc_v, scores_out.at[pl.ds(base, chunk)])

        pltpu.sync_copy(nm_v, partials.at[pl.ds(sid * n_pad, n_pad)])
        plsc.subcore_barrier()
        for t in range(ns):
            pltpu.sync_copy(partials.at[pl.ds(t * n_pad + sid * slc, slc)],
                            h_v.at[pl.ds(t * slc, slc)])

        def red_body(j, _):
            off = j * _L
            acc = h_v[pl.ds(off, _L)]
            for t in range(1, ns):
                acc = jnp.maximum(acc, h_v[pl.ds(t * slc + off, _L)])
            nm_v[pl.ds(off, _L)] = acc
            return 0

        lax.fori_loop(0, slc // _L, red_body, 0)
        pltpu.sync_copy(nm_v.at[pl.ds(0, slc)],
                        nm_out.at[pl.ds(cid * n_pad + sid * slc, slc)])

    @functools.partial(
        pl.kernel,
        out_type=jax.ShapeDtypeStruct((e_pad,), jnp.float32),
        mesh=mesh,
        compiler_params=pltpu.CompilerParams(needs_layout_passes=False),
        scratch_types=[
            pltpu.VMEM((n_pad,), jnp.float32),
            pltpu.VMEM((n_pad,), jnp.float32),
            pltpu.VMEM((chunk,), jnp.int32),
            pltpu.VMEM((chunk,), jnp.int32),
            pltpu.VMEM((chunk,), jnp.float32),
        ],
    )
    def sc2(nm_parts, src, dst, scores, out, nm_v, nm2_v, src_v, dst_v, sc_v):
        cid = lax.axis_index("c")
        sid = lax.axis_index("s")
        wid = sid * nc + cid
        base = wid * chunk

        pltpu.sync_copy(nm_parts.at[pl.ds(0, n_pad)], nm_v)
        pltpu.sync_copy(nm_parts.at[pl.ds(n_pad, n_pad)], nm2_v)
        pltpu.sync_copy(src.at[pl.ds(base, chunk)], src_v)
        pltpu.sync_copy(dst.at[pl.ds(base, chunk)], dst_v)
        pltpu.sync_copy(scores.at[pl.ds(base, chunk)], sc_v)

        def merge_body(j, _):
            sl = pl.ds(j * _L, _L)
            nm_v[sl] = jnp.maximum(nm_v[sl], nm2_v[sl])
            return 0

        lax.fori_loop(0, n_pad // _L, merge_body, 0)

        def sel_body(j, _):
            sl = pl.ds(j * _L, _L)
            s = sc_v[sl]
            ms = plsc.load_gather(nm_v, [src_v[sl]])
            md = plsc.load_gather(nm_v, [dst_v[sl]])
            keep = (s > 0.0) & (s >= ms) & (s >= md)
            sc_v[sl] = jnp.where(keep, s, 0.0)
            return 0

        lax.fori_loop(0, chunk // _L, sel_body, 0)
        pltpu.sync_copy(sc_v, out.at[pl.ds(base, chunk)])

    return sc1, sc2, n_pad, e_pad


@jax.jit
def kernel(node_logits, comb_logits, edge_index):
    n_nodes = node_logits.shape[0]
    n_edges = comb_logits.shape[0]
    sc1, sc2, n_pad, e_pad = _make_sc_kernels(n_nodes, n_edges)

    tot = n_pad + e_pad
    rows = (tot + 127) // 128
    tot_pad = rows * 128

    all_l = jnp.concatenate([
        node_logits,
        jnp.zeros((n_pad - n_nodes, 2), jnp.float32),
        comb_logits,
        jnp.zeros((tot_pad - n_pad - n_edges, 2), jnp.float32),
    ])
    l0 = all_l[:, 0].reshape(rows, 128)
    l1 = all_l[:, 1].reshape(rows, 128)

    h2d = pl.pallas_call(
        _entropy_tc_body,
        out_shape=jax.ShapeDtypeStruct((rows, 128), jnp.float32),
    )(l0, l1)
    h_all = h2d.reshape(tot_pad)

    src = jnp.pad(edge_index[0], (0, e_pad - n_edges))
    dst = jnp.pad(edge_index[1], (0, e_pad - n_edges))
    scores, nm_parts = sc1(h_all, src, dst)
    out = sc2(nm_parts, src, dst, scores)
    return out[:n_edges]
